# baseline (speedup 1.0000x reference)
import functools

import jax
import jax.numpy as jnp
from jax import lax
from jax.experimental import pallas as pl
from jax.experimental.pallas import tpu as pltpu
from jax.experimental.pallas import tpu_sc as plsc


def _edge_kernel(u_ref, w_ref, b_ref, s_ref, y_ref, *, blk_e):
    i = pl.program_id(0)
    u = u_ref[...]
    w = w_ref[...]
    z = jax.lax.dot_general(
        u.astype(jnp.bfloat16), w.astype(jnp.bfloat16),
        (((1,), (1,)), ((), ())),
        preferred_element_type=jnp.float32,
    )
    s = s_ref[0, 0]
    dyn = jax.nn.sigmoid(z + b_ref[0, :][None, :])
    u_blk = u_ref[:, pl.ds(i * blk_e, blk_e)]
    y_ref[...] = s * u_blk + (1.0 - s) * dyn


def _make_sc_broadcast(n_idx, seq_len):
    info = plsc.get_sparse_core_info()
    nc, ns, lanes = info.num_cores, info.num_subcores, info.num_lanes
    nw = nc * ns
    chunk = n_idx // nw
    n_grp = chunk // lanes
    n_rep = seq_len // lanes
    mesh = plsc.VectorSubcoreMesh(core_axis_name="c", subcore_axis_name="s")

    @functools.partial(
        pl.kernel,
        out_type=jax.ShapeDtypeStruct((n_idx * seq_len,), jnp.int32),
        mesh=mesh,
        scratch_types=[
            pltpu.VMEM((chunk,), jnp.int32),
            pltpu.VMEM((chunk * seq_len,), jnp.int32),
        ],
    )
    def sc_broadcast(idx_hbm, out_hbm, in_v, out_v):
        wid = lax.axis_index("s") * nc + lax.axis_index("c")
        base = wid * chunk
        pltpu.sync_copy(idx_hbm.at[pl.ds(base, chunk)], in_v)

        def body(g, carry):
            vals = in_v[pl.ds(g * lanes, lanes)]
            dnums = lax.GatherDimensionNumbers(
                offset_dims=(), collapsed_slice_dims=(0,), start_index_map=(0,))
            for i in range(lanes):
                sp = lax.gather(vals, jnp.full((lanes, 1), i, jnp.int32),
                                dnums, (1,),
                                mode=lax.GatherScatterMode.PROMISE_IN_BOUNDS)
                row = g * lanes + i
                for k in range(n_rep):
                    out_v[pl.ds(row * seq_len + k * lanes, lanes)] = sp
            return carry

        lax.fori_loop(0, n_grp, body, 0)
        pltpu.sync_copy(out_v, out_hbm.at[pl.ds(base * seq_len, chunk * seq_len)])

    return sc_broadcast


def kernel(hidden_states, edge_index, edge_weight, W, b, skip_param):
    seq_len = hidden_states.shape[1]
    E = W.shape[0]
    BE = edge_weight.shape[0]
    batch = BE // E

    u = edge_weight.reshape(batch, E)
    b2 = b.reshape(1, E)
    s2 = skip_param.reshape(1, 1)

    blk_e = 512
    n_blk = E // blk_e

    body = functools.partial(_edge_kernel, blk_e=blk_e)

    y2 = pl.pallas_call(
        body,
        grid=(n_blk,),
        in_specs=[
            pl.BlockSpec((batch, E), lambda i: (0, 0)),
            pl.BlockSpec((blk_e, E), lambda i: (i, 0)),
            pl.BlockSpec((1, blk_e), lambda i: (0, i)),
            pl.BlockSpec((1, 1), lambda i: (0, 0)),
        ],
        out_specs=pl.BlockSpec((batch, blk_e), lambda i: (0, i)),
        out_shape=jax.ShapeDtypeStruct((batch, E), jnp.float32),
    )(u, W, b2, s2)

    sc_broadcast = _make_sc_broadcast(2 * BE, seq_len)
    ei3 = sc_broadcast(edge_index.reshape(2 * BE)).reshape(2, BE, seq_len)


    out = jnp.broadcast_to(y2.reshape(BE, 1), (BE, seq_len))
    return ei3, out


# speedup vs baseline: 1.1124x; 1.1124x over previous
import functools

import jax
import jax.numpy as jnp
from jax.experimental import pallas as pl


def _edge_kernel(u_ref, w_ref, b_ref, s_ref, ei_ref, y_ref, ei3_ref, *,
                 blk_e, blk_i, seq_len, pair):
    i = pl.program_id(0)
    u = u_ref[...]
    w = w_ref[...]
    batch = u.shape[0]
    z = jax.lax.dot_general(
        u.astype(jnp.bfloat16), w.astype(jnp.bfloat16),
        (((1,), (1,)), ((), ())),
        preferred_element_type=jnp.float32,
    )
    s = s_ref[0, 0]
    dyn = jax.nn.sigmoid(z + b_ref[0, :][None, :])
    u_blk = u_ref[:, pl.ds(i * blk_e, blk_e)]
    y = s * u_blk + (1.0 - s) * dyn

    ya = y.reshape(batch, blk_e // pair, pair, 1)
    yb = jnp.broadcast_to(ya, (batch, blk_e // pair, pair, seq_len))
    y_ref[...] = yb.reshape(batch, blk_e // pair, pair * seq_len)

    ei = ei_ref[...]
    ea = ei.reshape(2, blk_i // pair, pair, 1)
    eb = jnp.broadcast_to(ea, (2, blk_i // pair, pair, seq_len))
    ei3_ref[...] = eb.reshape(2, blk_i // pair, pair * seq_len)


def kernel(hidden_states, edge_index, edge_weight, W, b, skip_param):
    seq_len = hidden_states.shape[1]
    E = W.shape[0]
    BE = edge_weight.shape[0]
    batch = BE // E
    pair = max(1, 128 // seq_len)

    u = edge_weight.reshape(batch, E)
    b2 = b.reshape(1, E)
    s2 = skip_param.reshape(1, 1)

    blk_e = 512
    n_blk = E // blk_e
    blk_i = BE // n_blk

    body = functools.partial(_edge_kernel, blk_e=blk_e, blk_i=blk_i,
                             seq_len=seq_len, pair=pair)

    y3, ei3p = pl.pallas_call(
        body,
        grid=(n_blk,),
        in_specs=[
            pl.BlockSpec((batch, E), lambda i: (0, 0)),
            pl.BlockSpec((blk_e, E), lambda i: (i, 0)),
            pl.BlockSpec((1, blk_e), lambda i: (0, i)),
            pl.BlockSpec((1, 1), lambda i: (0, 0)),
            pl.BlockSpec((2, blk_i), lambda i: (0, i)),
        ],
        out_specs=[
            pl.BlockSpec((batch, blk_e // pair, pair * seq_len),
                         lambda i: (0, i, 0)),
            pl.BlockSpec((2, blk_i // pair, pair * seq_len),
                         lambda i: (0, i, 0)),
        ],
        out_shape=[
            jax.ShapeDtypeStruct((batch, E // pair, pair * seq_len),
                                 jnp.float32),
            jax.ShapeDtypeStruct((2, BE // pair, pair * seq_len), jnp.int32),
        ],
    )(u, W, b2, s2, edge_index)

    out = y3.reshape(BE, seq_len)
    ei3 = ei3p.reshape(2, BE, seq_len)
    return ei3, out


# speedup vs baseline: 1.9974x; 1.7956x over previous
import functools

import jax
import jax.numpy as jnp
from jax.experimental import pallas as pl


def _edge_kernel(u_ref, wt_ref, wb_ref, bt_ref, bb_ref, s_ref, yt_ref, yb_ref,
                 *, blk_e, half):
    i = pl.program_id(0)
    u = u_ref[...]
    ub = u.astype(jnp.bfloat16)
    s = s_ref[0, 0]
    for w_ref, y_ref, bias_ref, base in (
            (wt_ref, yt_ref, bt_ref, 0),
            (wb_ref, yb_ref, bb_ref, half)):
        z = jax.lax.dot_general(
            ub, w_ref[...].astype(jnp.bfloat16),
            (((1,), (1,)), ((), ())),
            preferred_element_type=jnp.float32,
        )
        dyn = jax.nn.sigmoid(z + bias_ref[0, :][None, :])
        u_blk = u_ref[:, pl.ds(base + i * blk_e, blk_e)]
        y_ref[...] = s * u_blk + (1.0 - s) * dyn


def kernel(hidden_states, edge_index, edge_weight, W, b, skip_param):
    seq_len = hidden_states.shape[1]
    E = W.shape[0]
    BE = edge_weight.shape[0]
    batch = BE // E
    half = E // 2

    u = edge_weight.reshape(batch, E)
    b2 = b.reshape(1, E)
    s2 = skip_param.reshape(1, 1)

    blk_e = 256
    n_blk = half // blk_e

    body = functools.partial(_edge_kernel, blk_e=blk_e, half=half)

    yt, yb = pl.pallas_call(
        body,
        grid=(n_blk,),
        in_specs=[
            pl.BlockSpec((batch, E), lambda i: (0, 0)),
            pl.BlockSpec((blk_e, E), lambda i: (i, 0)),
            pl.BlockSpec((blk_e, E), lambda i, n=n_blk: (i + n, 0)),
            pl.BlockSpec((1, blk_e), lambda i: (0, i)),
            pl.BlockSpec((1, blk_e), lambda i, n=n_blk: (0, i + n)),
            pl.BlockSpec((1, 1), lambda i: (0, 0)),
        ],
        out_specs=[
            pl.BlockSpec((batch, blk_e), lambda i: (0, i)),
            pl.BlockSpec((batch, blk_e), lambda i: (0, i)),
        ],
        out_shape=[
            jax.ShapeDtypeStruct((batch, half), jnp.float32),
            jax.ShapeDtypeStruct((batch, half), jnp.float32),
        ],
    )(u, W, W, b2, b2, s2)

    y2 = jnp.concatenate([yt, yb], axis=1)
    ei3 = jnp.broadcast_to(edge_index[:, :, None], (2, BE, seq_len))
    out = jnp.broadcast_to(y2.reshape(BE, 1), (BE, seq_len))
    return ei3, out
